# Initial kernel scaffold; baseline (speedup 1.0000x reference)
import functools

import jax
import jax.numpy as jnp
from jax import lax
from jax.experimental import pallas as pl
from jax.experimental.pallas import tpu as pltpu
from jax.experimental.pallas import tpu_sc as plsc

N = 10000
E = 320000
D = 128

NC = 2
NS = 16
NW = NC * NS
EPW = E // NW
W = 80
NWIN = EPW // W
ROWS_PER_SUB = N // NS


def _sc_agg(feat, src, dst, with_deg):
    mesh = plsc.VectorSubcoreMesh(core_axis_name="c", subcore_axis_name="s")
    out_type = [jax.ShapeDtypeStruct((NC, N, D), jnp.float32)]
    scratch = [
        pltpu.VMEM_SHARED((N, D), jnp.float32),
        pltpu.VMEM((W,), jnp.int32),
        pltpu.VMEM((W,), jnp.int32),
        pltpu.VMEM((W, D), jnp.float32),
        pltpu.SemaphoreType.DMA,
    ]
    if with_deg:
        out_type.append(jax.ShapeDtypeStruct((NC, N, 16), jnp.float32))
        scratch.insert(1, pltpu.VMEM_SHARED((N, 16), jnp.float32))
        scratch.insert(4, pltpu.VMEM((W, 16), jnp.float32))

    def body(*refs):
        if with_deg:
            (feat_h, src_h, dst_h, parts_h, degp_h,
             acc_sh, deg_sh, src_v, dst_v, ones_v, rows_v, sem) = refs
        else:
            (feat_h, src_h, dst_h, parts_h,
             acc_sh, src_v, dst_v, rows_v, sem) = refs
        c = lax.axis_index("c")
        s = lax.axis_index("s")
        wid = s * NC + c

        @pl.loop(0, W)
        def _(i):
            @pl.loop(0, D, step=16)
            def _(j):
                rows_v.at[i, pl.ds(j, 16)][...] = jnp.zeros((16,), jnp.float32)
            if with_deg:
                ones_v.at[i][...] = jnp.full((16,), 1.0, jnp.float32)

        @pl.loop(s, N // W, step=NS)
        def _(chunk):
            pltpu.sync_copy(rows_v, acc_sh.at[pl.ds(chunk * W, W)])

        if with_deg:
            @pl.loop(s, N // W, step=NS)
            def _(chunk):
                pltpu.sync_copy(rows_v.at[:, pl.ds(0, 16)],
                                deg_sh.at[pl.ds(chunk * W, W)])

        plsc.subcore_barrier()

        @pl.loop(0, NWIN)
        def _(w):
            base = wid * EPW + w * W
            pltpu.sync_copy(src_h.at[pl.ds(base, W)], src_v)
            pltpu.sync_copy(dst_h.at[pl.ds(base, W)], dst_v)
            pltpu.async_copy(feat_h.at[src_v], rows_v, sem).wait()
            pltpu.sync_copy(rows_v, acc_sh.at[dst_v], add=True)
            if with_deg:
                pltpu.sync_copy(ones_v, deg_sh.at[dst_v], add=True)

        plsc.subcore_barrier()

        lo = s * ROWS_PER_SUB
        pltpu.sync_copy(acc_sh.at[pl.ds(lo, ROWS_PER_SUB)],
                        parts_h.at[c].at[pl.ds(lo, ROWS_PER_SUB)])
        if with_deg:
            pltpu.sync_copy(deg_sh.at[pl.ds(lo, ROWS_PER_SUB)],
                            degp_h.at[c].at[pl.ds(lo, ROWS_PER_SUB)])

    fn = pl.kernel(body, out_type=tuple(out_type), mesh=mesh,
                   scratch_types=scratch)
    return fn(feat, src, dst)


_BN = 1000


def _tc_body(relu, x_ref, p_ref, d_ref, wa_ref, wb_ref, o_ref):
    d = d_ref[...]
    deg = d[0, :, 0:1] + d[1, :, 0:1]
    recip = 1.0 / jnp.maximum(deg, 1.0)
    agg = (p_ref[0] + p_ref[1]) * recip
    y = (jnp.dot(x_ref[...], wa_ref[...], preferred_element_type=jnp.float32)
         + jnp.dot(agg, wb_ref[...], preferred_element_type=jnp.float32))
    if relu:
        y = jnp.maximum(y, 0.0)
    o_ref[...] = y


def _tc_layer(x, parts, degp, wa_t, wb_t, relu):
    grid = (N // _BN,)
    return pl.pallas_call(
        functools.partial(_tc_body, relu),
        grid=grid,
        in_specs=[
            pl.BlockSpec((_BN, D), lambda i: (i, 0)),
            pl.BlockSpec((NC, _BN, D), lambda i: (0, i, 0)),
            pl.BlockSpec((NC, _BN, 16), lambda i: (0, i, 0)),
            pl.BlockSpec((D, D), lambda i: (0, 0)),
            pl.BlockSpec((D, D), lambda i: (0, 0)),
        ],
        out_specs=pl.BlockSpec((_BN, D), lambda i: (i, 0)),
        out_shape=jax.ShapeDtypeStruct((N, D), jnp.float32),
    )(x, parts, degp, wa_t, wb_t)


def kernel(feat, edge_index, W0, W1):
    src = edge_index[0]
    dst = edge_index[1]
    w0a_t = W0[:, :D].T
    w0b_t = W0[:, D:].T
    w1a_t = W1[:, :D].T
    w1b_t = W1[:, D:].T

    parts0, degp = _sc_agg(feat, src, dst, with_deg=True)
    h = _tc_layer(feat, parts0, degp, w0a_t, w0b_t, relu=True)
    parts1, = _sc_agg(h, src, dst, with_deg=False)
    out = _tc_layer(h, parts1, degp, w1a_t, w1b_t, relu=False)
    return out


# speedup vs baseline: 5.6433x; 5.6433x over previous
import dataclasses
import functools

import jax
import jax.numpy as jnp
from jax import lax
from jax.experimental import pallas as pl
from jax.experimental.pallas import tpu as pltpu
from jax.experimental.pallas import tpu_sc as plsc

N = 10000
E = 320000
D = 128

NC = 2
NS = 16
NW = NC * NS
EPW = E // NW
W = 80
NWIN = EPW // W
L = 16


def _sc_agg(feat, src, dst, with_deg):
    mesh = plsc.VectorSubcoreMesh(core_axis_name="c", subcore_axis_name="s")
    out_type = [jax.ShapeDtypeStruct((NC, N, D), jnp.float32)]
    scratch = [
        pltpu.VMEM_SHARED((N, D), jnp.float32),
        pltpu.VMEM((W,), jnp.int32),
        pltpu.VMEM((W,), jnp.int32),
        pltpu.VMEM((W, D), jnp.float32),
        pltpu.SemaphoreType.DMA,
    ]
    if with_deg:
        out_type.append(jax.ShapeDtypeStruct((NW, 1, N), jnp.float32))
        scratch.append(pltpu.VMEM((N,), jnp.float32))

    def body(*refs):
        if with_deg:
            (feat_h, src_h, dst_h, parts_h, degp_h,
             acc_sh, src_v, dst_v, rows_v, sem, hist_v) = refs
        else:
            (feat_h, src_h, dst_h, parts_h,
             acc_sh, src_v, dst_v, rows_v, sem) = refs
        c = lax.axis_index("c")
        s = lax.axis_index("s")
        wid = s * NC + c

        @pl.loop(0, W)
        def _(i):
            @pl.loop(0, D, step=L)
            def _(j):
                rows_v.at[i, pl.ds(j, L)][...] = jnp.zeros((L,), jnp.float32)

        if with_deg:
            @pl.loop(0, N, step=L)
            def _(i):
                hist_v.at[pl.ds(i, L)][...] = jnp.zeros((L,), jnp.float32)

        @pl.loop(s, N // W, step=NS)
        def _(chunk):
            pltpu.sync_copy(rows_v, acc_sh.at[pl.ds(chunk * W, W)])

        plsc.subcore_barrier()

        ones = jnp.full((L,), 1.0, jnp.float32)

        @pl.loop(0, NWIN)
        def _(w):
            base = wid * EPW + w * W
            pltpu.sync_copy(src_h.at[pl.ds(base, W)], src_v)
            pltpu.sync_copy(dst_h.at[pl.ds(base, W)], dst_v)
            pltpu.async_copy(feat_h.at[src_v], rows_v, sem).wait()
            pltpu.sync_copy(rows_v, acc_sh.at[dst_v], add=True)
            if with_deg:
                for k in range(W // L):
                    idx = dst_v[pl.ds(k * L, L)]
                    plsc.addupdate_scatter(hist_v, [idx], ones)

        plsc.subcore_barrier()

        @pl.loop(s, N // W, step=NS)
        def _(chunk):
            lo = chunk * W
            pltpu.sync_copy(acc_sh.at[pl.ds(lo, W)],
                            parts_h.at[c].at[pl.ds(lo, W)])

        if with_deg:
            pltpu.sync_copy(hist_v, degp_h.at[wid].at[0])

    cp = pltpu.CompilerParams()
    if "needs_layout_passes" in pltpu.CompilerParams.__dataclass_fields__:
        cp = dataclasses.replace(cp, needs_layout_passes=False)
    fn = pl.kernel(body, out_type=tuple(out_type), mesh=mesh,
                   scratch_types=scratch, compiler_params=cp)
    return fn(feat, src, dst)


def _recip_body(d_ref, r_ref):
    d = d_ref[...][:, 0, :]
    ones = jnp.ones((NW, 1), jnp.float32)
    deg = lax.dot_general(d, ones, (((0,), (0,)), ((), ())),
                          preferred_element_type=jnp.float32)
    r_ref[...] = jnp.broadcast_to(1.0 / jnp.maximum(deg, 1.0), (N, D))


def _tc_recip(degp):
    return pl.pallas_call(
        _recip_body,
        out_shape=jax.ShapeDtypeStruct((N, D), jnp.float32),
    )(degp)


_BN = 1000


def _tc_body(relu, x_ref, p_ref, r_ref, wa_ref, wb_ref, o_ref):
    agg = (p_ref[0] + p_ref[1]) * r_ref[...]
    y = (jnp.dot(x_ref[...], wa_ref[...], preferred_element_type=jnp.float32)
         + jnp.dot(agg, wb_ref[...], preferred_element_type=jnp.float32))
    if relu:
        y = jnp.maximum(y, 0.0)
    o_ref[...] = y


def _tc_layer(x, parts, recipb, wa_t, wb_t, relu):
    grid = (N // _BN,)
    return pl.pallas_call(
        functools.partial(_tc_body, relu),
        grid=grid,
        in_specs=[
            pl.BlockSpec((_BN, D), lambda i: (i, 0)),
            pl.BlockSpec((NC, _BN, D), lambda i: (0, i, 0)),
            pl.BlockSpec((_BN, D), lambda i: (i, 0)),
            pl.BlockSpec((D, D), lambda i: (0, 0)),
            pl.BlockSpec((D, D), lambda i: (0, 0)),
        ],
        out_specs=pl.BlockSpec((_BN, D), lambda i: (i, 0)),
        out_shape=jax.ShapeDtypeStruct((N, D), jnp.float32),
    )(x, parts, recipb, wa_t, wb_t)


def kernel(feat, edge_index, W0, W1):
    src = edge_index[0]
    dst = edge_index[1]
    w0a_t = W0[:, :D].T
    w0b_t = W0[:, D:].T
    w1a_t = W1[:, :D].T
    w1b_t = W1[:, D:].T

    parts0, degp = _sc_agg(feat, src, dst, with_deg=True)
    recipb = _tc_recip(degp)
    h = _tc_layer(feat, parts0, recipb, w0a_t, w0b_t, relu=True)
    parts1, = _sc_agg(h, src, dst, with_deg=False)
    out = _tc_layer(h, parts1, recipb, w1a_t, w1b_t, relu=False)
    return out


# speedup vs baseline: 11.4102x; 2.0219x over previous
import dataclasses
import functools

import jax
import jax.numpy as jnp
from jax import lax
from jax.experimental import pallas as pl
from jax.experimental.pallas import tpu as pltpu
from jax.experimental.pallas import tpu_sc as plsc

N = 10000
E = 320000
D = 128

NC = 2
NS = 16
NW = NC * NS
EPW = E // NW
W = 80
NWIN = EPW // W
K = 5
NBUF = 3
L = 16


def _sc_agg(feat, src, dst, with_deg):
    mesh = plsc.VectorSubcoreMesh(core_axis_name="c", subcore_axis_name="s")
    out_type = [jax.ShapeDtypeStruct((NC, N, D), jnp.float32)]
    scratch = [
        pltpu.VMEM_SHARED((N, D), jnp.float32),
        pltpu.VMEM((K * W,), jnp.int32),
        pltpu.VMEM((K * W,), jnp.int32),
        pltpu.VMEM((W,), jnp.int32),
    ] + [pltpu.VMEM((W, D), jnp.float32) for _ in range(NBUF)] + [
        pltpu.SemaphoreType.DMA,
    ]
    if with_deg:
        out_type.append(jax.ShapeDtypeStruct((NW, 1, N), jnp.float32))
        scratch.append(pltpu.VMEM((N,), jnp.float32))

    def body(*refs):
        if with_deg:
            (feat_h, src_h, dst_h, parts_h, degp_h,
             acc_sh, src_v, dst_v, dstw_v, *rest) = refs
            rows = rest[:NBUF]
            sem = rest[NBUF]
            hist_v = rest[NBUF + 1]
        else:
            (feat_h, src_h, dst_h, parts_h,
             acc_sh, src_v, dst_v, dstw_v, *rest) = refs
            rows = rest[:NBUF]
            sem = rest[NBUF]
        c = lax.axis_index("c")
        s = lax.axis_index("s")
        wid = s * NC + c

        @pl.loop(0, W)
        def _(i):
            @pl.loop(0, D, step=L)
            def _(j):
                rows[0].at[i, pl.ds(j, L)][...] = jnp.zeros((L,), jnp.float32)

        if with_deg:
            @pl.loop(0, N, step=L)
            def _(i):
                hist_v.at[pl.ds(i, L)][...] = jnp.zeros((L,), jnp.float32)

        @pl.loop(s, N // W, step=NS)
        def _(chunk):
            pltpu.sync_copy(rows[0], acc_sh.at[pl.ds(chunk * W, W)])

        plsc.subcore_barrier()

        ones = jnp.full((L,), 1.0, jnp.float32)

        def do_scatter(j, buf):
            for k in range(W // L):
                v = dst_v[pl.ds(j * W + k * L, L)]
                dstw_v.at[pl.ds(k * L, L)][...] = v
                if with_deg:
                    plsc.addupdate_scatter(hist_v, [v], ones)
            pltpu.sync_copy(buf, acc_sh.at[dstw_v], add=True)

        def fire(j):
            return pltpu.async_copy(
                feat_h.at[src_v.at[pl.ds(j * W, W)]], rows[j % NBUF], sem)

        @pl.loop(0, NWIN // K)
        def _(g):
            base = wid * EPW + g * K * W
            pltpu.sync_copy(src_h.at[pl.ds(base, K * W)], src_v)
            pltpu.sync_copy(dst_h.at[pl.ds(base, K * W)], dst_v)
            descs = [fire(j) for j in range(NBUF)]
            for j in range(K):
                b = j % NBUF
                descs[b].wait()
                do_scatter(j, rows[b])
                if j + NBUF < K:
                    descs[b] = fire(j + NBUF)

        plsc.subcore_barrier()

        @pl.loop(s, N // W, step=NS)
        def _(chunk):
            lo = chunk * W
            pltpu.sync_copy(acc_sh.at[pl.ds(lo, W)],
                            parts_h.at[c].at[pl.ds(lo, W)])

        if with_deg:
            pltpu.sync_copy(hist_v, degp_h.at[wid].at[0])

    cp = pltpu.CompilerParams()
    if "needs_layout_passes" in pltpu.CompilerParams.__dataclass_fields__:
        cp = dataclasses.replace(cp, needs_layout_passes=False)
    fn = pl.kernel(body, out_type=tuple(out_type), mesh=mesh,
                   scratch_types=scratch, compiler_params=cp)
    return fn(feat, src, dst)


def _recip_body(d_ref, r_ref):
    d = d_ref[...][:, 0, :]
    ones = jnp.ones((NW, 1), jnp.float32)
    deg = lax.dot_general(d, ones, (((0,), (0,)), ((), ())),
                          preferred_element_type=jnp.float32)
    r_ref[...] = jnp.broadcast_to(1.0 / jnp.maximum(deg, 1.0), (N, D))


def _tc_recip(degp):
    return pl.pallas_call(
        _recip_body,
        out_shape=jax.ShapeDtypeStruct((N, D), jnp.float32),
    )(degp)


_BN = 1000


def _tc_body(relu, x_ref, p_ref, r_ref, wa_ref, wb_ref, o_ref):
    agg = (p_ref[0] + p_ref[1]) * r_ref[...]
    y = (jnp.dot(x_ref[...], wa_ref[...], preferred_element_type=jnp.float32)
         + jnp.dot(agg, wb_ref[...], preferred_element_type=jnp.float32))
    if relu:
        y = jnp.maximum(y, 0.0)
    o_ref[...] = y


def _tc_layer(x, parts, recipb, wa_t, wb_t, relu):
    grid = (N // _BN,)
    return pl.pallas_call(
        functools.partial(_tc_body, relu),
        grid=grid,
        in_specs=[
            pl.BlockSpec((_BN, D), lambda i: (i, 0)),
            pl.BlockSpec((NC, _BN, D), lambda i: (0, i, 0)),
            pl.BlockSpec((_BN, D), lambda i: (i, 0)),
            pl.BlockSpec((D, D), lambda i: (0, 0)),
            pl.BlockSpec((D, D), lambda i: (0, 0)),
        ],
        out_specs=pl.BlockSpec((_BN, D), lambda i: (i, 0)),
        out_shape=jax.ShapeDtypeStruct((N, D), jnp.float32),
    )(x, parts, recipb, wa_t, wb_t)


def kernel(feat, edge_index, W0, W1):
    src = edge_index[0]
    dst = edge_index[1]
    w0a_t = W0[:, :D].T
    w0b_t = W0[:, D:].T
    w1a_t = W1[:, :D].T
    w1b_t = W1[:, D:].T

    parts0, degp = _sc_agg(feat, src, dst, with_deg=True)
    recipb = _tc_recip(degp)
    h = _tc_layer(feat, parts0, recipb, w0a_t, w0b_t, relu=True)
    parts1, = _sc_agg(h, src, dst, with_deg=False)
    out = _tc_layer(h, parts1, recipb, w1a_t, w1b_t, relu=False)
    return out


# speedup vs baseline: 14.2498x; 1.2489x over previous
import dataclasses
import functools

import jax
import jax.numpy as jnp
from jax import lax
from jax.experimental import pallas as pl
from jax.experimental.pallas import tpu as pltpu
from jax.experimental.pallas import tpu_sc as plsc

N = 10000
E = 320000
D = 128

NC = 2
NS = 16
NW = NC * NS
EPW = E // NW
W = 80
NWIN = EPW // W
K = 25
NBUF = 3
L = 16


def _sc_agg(feat, src, dst, with_deg):
    mesh = plsc.VectorSubcoreMesh(core_axis_name="c", subcore_axis_name="s")
    out_type = [jax.ShapeDtypeStruct((NC, N, D), jnp.float32)]
    scratch = [
        pltpu.VMEM_SHARED((N, D), jnp.float32),
        pltpu.VMEM((K * W,), jnp.int32),
        pltpu.VMEM((K * W,), jnp.int32),
        pltpu.VMEM((W,), jnp.int32),
    ] + [pltpu.VMEM((W, D), jnp.float32) for _ in range(NBUF)] + [
        pltpu.SemaphoreType.DMA,
    ]
    if with_deg:
        out_type.append(jax.ShapeDtypeStruct((NW, 1, N), jnp.float32))
        scratch.append(pltpu.VMEM((N,), jnp.float32))

    def body(*refs):
        if with_deg:
            (feat_h, src_h, dst_h, parts_h, degp_h,
             acc_sh, src_v, dst_v, dstw_v, *rest) = refs
            rows = rest[:NBUF]
            sem = rest[NBUF]
            hist_v = rest[NBUF + 1]
        else:
            (feat_h, src_h, dst_h, parts_h,
             acc_sh, src_v, dst_v, dstw_v, *rest) = refs
            rows = rest[:NBUF]
            sem = rest[NBUF]
        c = lax.axis_index("c")
        s = lax.axis_index("s")
        wid = s * NC + c

        @pl.loop(0, W)
        def _(i):
            @pl.loop(0, D, step=L)
            def _(j):
                rows[0].at[i, pl.ds(j, L)][...] = jnp.zeros((L,), jnp.float32)

        if with_deg:
            @pl.loop(0, N, step=L)
            def _(i):
                hist_v.at[pl.ds(i, L)][...] = jnp.zeros((L,), jnp.float32)

        @pl.loop(s, N // W, step=NS)
        def _(chunk):
            pltpu.sync_copy(rows[0], acc_sh.at[pl.ds(chunk * W, W)])

        plsc.subcore_barrier()

        ones = jnp.full((L,), 1.0, jnp.float32)

        def do_scatter(j, buf):
            for k in range(W // L):
                v = dst_v[pl.ds(j * W + k * L, L)]
                dstw_v.at[pl.ds(k * L, L)][...] = v
                if with_deg:
                    plsc.addupdate_scatter(hist_v, [v], ones)
            pltpu.sync_copy(buf, acc_sh.at[dstw_v], add=True)

        def fire(j):
            return pltpu.async_copy(
                feat_h.at[src_v.at[pl.ds(j * W, W)]], rows[j % NBUF], sem)

        @pl.loop(0, NWIN // K)
        def _(g):
            base = wid * EPW + g * K * W
            pltpu.sync_copy(src_h.at[pl.ds(base, K * W)], src_v)
            pltpu.sync_copy(dst_h.at[pl.ds(base, K * W)], dst_v)
            descs = [fire(j) for j in range(NBUF)]
            for j in range(K):
                b = j % NBUF
                descs[b].wait()
                do_scatter(j, rows[b])
                if j + NBUF < K:
                    descs[b] = fire(j + NBUF)

        plsc.subcore_barrier()

        @pl.loop(s, N // W, step=NS)
        def _(chunk):
            lo = chunk * W
            pltpu.sync_copy(acc_sh.at[pl.ds(lo, W)],
                            parts_h.at[c].at[pl.ds(lo, W)])

        if with_deg:
            pltpu.sync_copy(hist_v, degp_h.at[wid].at[0])

    cp = pltpu.CompilerParams()
    if "needs_layout_passes" in pltpu.CompilerParams.__dataclass_fields__:
        cp = dataclasses.replace(cp, needs_layout_passes=False)
    fn = pl.kernel(body, out_type=tuple(out_type), mesh=mesh,
                   scratch_types=scratch, compiler_params=cp)
    return fn(feat, src, dst)


def _recip_body(d_ref, r_ref):
    d = d_ref[...][:, 0, :]
    ones = jnp.ones((NW, 1), jnp.float32)
    deg = lax.dot_general(d, ones, (((0,), (0,)), ((), ())),
                          preferred_element_type=jnp.float32)
    r_ref[...] = jnp.broadcast_to(1.0 / jnp.maximum(deg, 1.0), (N, D))


def _tc_recip(degp):
    return pl.pallas_call(
        _recip_body,
        out_shape=jax.ShapeDtypeStruct((N, D), jnp.float32),
    )(degp)


_BN = 1000


def _tc_body(relu, x_ref, p_ref, r_ref, wa_ref, wb_ref, o_ref):
    agg = (p_ref[0] + p_ref[1]) * r_ref[...]
    y = (jnp.dot(x_ref[...], wa_ref[...], preferred_element_type=jnp.float32)
         + jnp.dot(agg, wb_ref[...], preferred_element_type=jnp.float32))
    if relu:
        y = jnp.maximum(y, 0.0)
    o_ref[...] = y


def _tc_layer(x, parts, recipb, wa_t, wb_t, relu):
    grid = (N // _BN,)
    return pl.pallas_call(
        functools.partial(_tc_body, relu),
        grid=grid,
        in_specs=[
            pl.BlockSpec((_BN, D), lambda i: (i, 0)),
            pl.BlockSpec((NC, _BN, D), lambda i: (0, i, 0)),
            pl.BlockSpec((_BN, D), lambda i: (i, 0)),
            pl.BlockSpec((D, D), lambda i: (0, 0)),
            pl.BlockSpec((D, D), lambda i: (0, 0)),
        ],
        out_specs=pl.BlockSpec((_BN, D), lambda i: (i, 0)),
        out_shape=jax.ShapeDtypeStruct((N, D), jnp.float32),
    )(x, parts, recipb, wa_t, wb_t)


def kernel(feat, edge_index, W0, W1):
    src = edge_index[0]
    dst = edge_index[1]
    w0a_t = W0[:, :D].T
    w0b_t = W0[:, D:].T
    w1a_t = W1[:, :D].T
    w1b_t = W1[:, D:].T

    parts0, degp = _sc_agg(feat, src, dst, with_deg=True)
    recipb = _tc_recip(degp)
    h = _tc_layer(feat, parts0, recipb, w0a_t, w0b_t, relu=True)
    parts1, = _sc_agg(h, src, dst, with_deg=False)
    out = _tc_layer(h, parts1, recipb, w1a_t, w1b_t, relu=False)
    return out


# speedup vs baseline: 14.4796x; 1.0161x over previous
import dataclasses
import functools

import jax
import jax.numpy as jnp
from jax import lax
from jax.experimental import pallas as pl
from jax.experimental.pallas import tpu as pltpu
from jax.experimental.pallas import tpu_sc as plsc

N = 10000
E = 320000
D = 128

NC = 2
NS = 16
NW = NC * NS
EPW = E // NW
W = 80
NWIN = EPW // W
K = 25
NBUF = 3
L = 16


def _sc_agg(feat, src, dst, with_deg):
    mesh = plsc.VectorSubcoreMesh(core_axis_name="c", subcore_axis_name="s")
    out_type = [jax.ShapeDtypeStruct((NC, N, D), jnp.float32)]
    scratch = [
        pltpu.VMEM_SHARED((N, D), jnp.float32),
        pltpu.VMEM((K * W,), jnp.int32),
        pltpu.VMEM((K * W,), jnp.int32),
        pltpu.VMEM((W,), jnp.int32),
    ] + [pltpu.VMEM((W, D), jnp.float32) for _ in range(NBUF)] + [
        pltpu.SemaphoreType.DMA,
    ]
    if with_deg:
        out_type.append(jax.ShapeDtypeStruct((NW, 1, N), jnp.float32))
        scratch.append(pltpu.VMEM((N,), jnp.float32))

    def body(*refs):
        if with_deg:
            (feat_h, src_h, dst_h, parts_h, degp_h,
             acc_sh, src_v, dst_v, dstw_v, *rest) = refs
            rows = rest[:NBUF]
            sem = rest[NBUF]
            hist_v = rest[NBUF + 1]
        else:
            (feat_h, src_h, dst_h, parts_h,
             acc_sh, src_v, dst_v, dstw_v, *rest) = refs
            rows = rest[:NBUF]
            sem = rest[NBUF]
        c = lax.axis_index("c")
        s = lax.axis_index("s")
        wid = s * NC + c

        @pl.loop(0, W)
        def _(i):
            @pl.loop(0, D, step=L)
            def _(j):
                rows[0].at[i, pl.ds(j, L)][...] = jnp.zeros((L,), jnp.float32)

        if with_deg:
            @pl.loop(0, N, step=L)
            def _(i):
                hist_v.at[pl.ds(i, L)][...] = jnp.zeros((L,), jnp.float32)

        nfull = (N // W) // NS

        def for_my_chunks(fn):
            for k in range(nfull):
                fn((s + k * NS) * W)
            tail = s + nfull * NS

            @pl.when(tail < N // W)
            def _():
                fn(tail * W)

        for_my_chunks(lambda lo: pltpu.async_copy(
            rows[0], acc_sh.at[pl.ds(lo, W)], sem))
        for_my_chunks(lambda lo: pltpu.make_async_copy(
            rows[0], acc_sh.at[pl.ds(lo, W)], sem).wait())

        plsc.subcore_barrier()

        ones = jnp.full((L,), 1.0, jnp.float32)

        def do_scatter(j, buf):
            for k in range(W // L):
                v = dst_v[pl.ds(j * W + k * L, L)]
                dstw_v.at[pl.ds(k * L, L)][...] = v
                if with_deg:
                    plsc.addupdate_scatter(hist_v, [v], ones)
            pltpu.sync_copy(buf, acc_sh.at[dstw_v], add=True)

        def fire(j):
            return pltpu.async_copy(
                feat_h.at[src_v.at[pl.ds(j * W, W)]], rows[j % NBUF], sem)

        @pl.loop(0, NWIN // K)
        def _(g):
            base = wid * EPW + g * K * W
            pltpu.sync_copy(src_h.at[pl.ds(base, K * W)], src_v)
            pltpu.sync_copy(dst_h.at[pl.ds(base, K * W)], dst_v)
            descs = [fire(j) for j in range(NBUF)]
            for j in range(K):
                b = j % NBUF
                descs[b].wait()
                do_scatter(j, rows[b])
                if j + NBUF < K:
                    descs[b] = fire(j + NBUF)

        plsc.subcore_barrier()

        for_my_chunks(lambda lo: pltpu.async_copy(
            acc_sh.at[pl.ds(lo, W)], parts_h.at[c].at[pl.ds(lo, W)], sem))
        if with_deg:
            outd = pltpu.async_copy(hist_v, degp_h.at[wid].at[0], sem)
        for_my_chunks(lambda lo: pltpu.make_async_copy(
            acc_sh.at[pl.ds(lo, W)], parts_h.at[c].at[pl.ds(lo, W)], sem).wait())
        if with_deg:
            outd.wait()

    cp = pltpu.CompilerParams()
    if "needs_layout_passes" in pltpu.CompilerParams.__dataclass_fields__:
        cp = dataclasses.replace(cp, needs_layout_passes=False)
    fn = pl.kernel(body, out_type=tuple(out_type), mesh=mesh,
                   scratch_types=scratch, compiler_params=cp)
    return fn(feat, src, dst)


_BN = 1000


def _tc0_body(x_ref, p_ref, d_ref, wa_ref, wb_ref, o_ref, r_out_ref, recip_s):
    i = pl.program_id(0)

    @pl.when(i == 0)
    def _():
        d = d_ref[...][:, 0, :]
        ones = jnp.ones((NW, 1), jnp.float32)
        deg = lax.dot_general(d, ones, (((0,), (0,)), ((), ())),
                              preferred_element_type=jnp.float32)
        recip_s[...] = jnp.broadcast_to(1.0 / jnp.maximum(deg, 1.0), (N, D))

    r = recip_s[pl.ds(i * _BN, _BN), :]
    r_out_ref[...] = r
    agg = (p_ref[0] + p_ref[1]) * r
    y = (jnp.dot(x_ref[...], wa_ref[...], preferred_element_type=jnp.float32)
         + jnp.dot(agg, wb_ref[...], preferred_element_type=jnp.float32))
    o_ref[...] = jnp.maximum(y, 0.0)


def _tc_layer0(x, parts, degp, wa_t, wb_t):
    return pl.pallas_call(
        _tc0_body,
        grid=(N // _BN,),
        in_specs=[
            pl.BlockSpec((_BN, D), lambda i: (i, 0)),
            pl.BlockSpec((NC, _BN, D), lambda i: (0, i, 0)),
            pl.BlockSpec((NW, 1, N), lambda i: (0, 0, 0)),
            pl.BlockSpec((D, D), lambda i: (0, 0)),
            pl.BlockSpec((D, D), lambda i: (0, 0)),
        ],
        out_specs=[
            pl.BlockSpec((_BN, D), lambda i: (i, 0)),
            pl.BlockSpec((_BN, D), lambda i: (i, 0)),
        ],
        out_shape=[
            jax.ShapeDtypeStruct((N, D), jnp.float32),
            jax.ShapeDtypeStruct((N, D), jnp.float32),
        ],
        scratch_shapes=[pltpu.VMEM((N, D), jnp.float32)],
    )(x, parts, degp, wa_t, wb_t)


def _tc_body(relu, x_ref, p_ref, r_ref, wa_ref, wb_ref, o_ref):
    agg = (p_ref[0] + p_ref[1]) * r_ref[...]
    y = (jnp.dot(x_ref[...], wa_ref[...], preferred_element_type=jnp.float32)
         + jnp.dot(agg, wb_ref[...], preferred_element_type=jnp.float32))
    if relu:
        y = jnp.maximum(y, 0.0)
    o_ref[...] = y


def _tc_layer(x, parts, recipb, wa_t, wb_t, relu):
    grid = (N // _BN,)
    return pl.pallas_call(
        functools.partial(_tc_body, relu),
        grid=grid,
        in_specs=[
            pl.BlockSpec((_BN, D), lambda i: (i, 0)),
            pl.BlockSpec((NC, _BN, D), lambda i: (0, i, 0)),
            pl.BlockSpec((_BN, D), lambda i: (i, 0)),
            pl.BlockSpec((D, D), lambda i: (0, 0)),
            pl.BlockSpec((D, D), lambda i: (0, 0)),
        ],
        out_specs=pl.BlockSpec((_BN, D), lambda i: (i, 0)),
        out_shape=jax.ShapeDtypeStruct((N, D), jnp.float32),
    )(x, parts, recipb, wa_t, wb_t)


def kernel(feat, edge_index, W0, W1):
    src = edge_index[0]
    dst = edge_index[1]
    w0a_t = W0[:, :D].T
    w0b_t = W0[:, D:].T
    w1a_t = W1[:, :D].T
    w1b_t = W1[:, D:].T

    parts0, degp = _sc_agg(feat, src, dst, with_deg=True)
    h, recipb = _tc_layer0(feat, parts0, degp, w0a_t, w0b_t)
    parts1, = _sc_agg(h, src, dst, with_deg=False)
    out = _tc_layer(h, parts1, recipb, w1a_t, w1b_t, relu=False)
    return out
